# baseline (speedup 1.0000x reference)
import jax
import jax.numpy as jnp
from jax import lax
from jax.experimental import pallas as pl
from jax.experimental.pallas import tpu as pltpu
from jax.experimental.pallas import tpu_sc as plsc

N_CLASSES = 1000
N_UNITS = 128
BATCH = 16384

NC = 2
NS = 16
LANES = 16
NW = NC * NS
B_PER_W = BATCH // NW
CHUNK = 32
NCHUNK = B_PER_W // CHUNK
RING = 4
PREFETCH = 3


def _gate_kernel(x_hbm, label_hbm, table_hbm, out_hbm,
                 idx_v, rows_v, x_v, table_sh, gsem, lsem, ssem):
    sid = lax.axis_index("s")
    wid = sid * NC + lax.axis_index("c")
    base = wid * B_PER_W

    for c in range(PREFETCH):
        pltpu.async_copy(
            x_hbm.at[pl.ds(base + c * CHUNK, CHUNK)],
            x_v.at[pl.ds(c * CHUNK, CHUNK)], lsem)

    @pl.when(sid < 7)
    def _stage():
        r0 = sid * 128
        pltpu.sync_copy(table_hbm.at[pl.ds(r0, 128)],
                        table_sh.at[pl.ds(r0, 128)])

    @pl.when(sid == 7)
    def _stage_tail():
        pltpu.sync_copy(table_hbm.at[pl.ds(896, N_CLASSES - 896)],
                        table_sh.at[pl.ds(896, N_CLASSES - 896)])

    pltpu.sync_copy(label_hbm.at[pl.ds(base, B_PER_W)], idx_v)
    plsc.subcore_barrier()

    for c in range(NCHUNK):
        pltpu.async_copy(
            table_sh.at[idx_v.at[pl.ds(c * CHUNK, CHUNK)]],
            rows_v.at[pl.ds(c * CHUNK, CHUNK)], gsem)

    def chunk(c, carry):
        xoff = pl.multiple_of(lax.rem(c, RING) * CHUNK, CHUNK)
        roff = pl.multiple_of(c * CHUNK, CHUNK)
        pltpu.make_async_copy(
            table_sh.at[idx_v.at[pl.ds(roff, CHUNK)]],
            rows_v.at[pl.ds(roff, CHUNK)], gsem).wait()
        pltpu.make_async_copy(
            x_hbm.at[pl.ds(base, CHUNK)], x_v.at[pl.ds(xoff, CHUNK)], lsem).wait()

        @plsc.parallel_loop(0, CHUNK, unroll=4)
        def body(r):
            for j in range(N_UNITS // LANES):
                sl = pl.ds(j * LANES, LANES)
                x_v[xoff + r, sl] = x_v[xoff + r, sl] * rows_v[roff + r, sl]

        @pl.when(jnp.logical_and(c >= 1, c + PREFETCH < NCHUNK))
        def _drain_store():
            pltpu.make_async_copy(
                x_v.at[pl.ds(0, CHUNK)], out_hbm.at[pl.ds(base, CHUNK)],
                ssem).wait()

        @pl.when(c + PREFETCH < NCHUNK)
        def _next_load():
            off2 = pl.multiple_of(lax.rem(c + PREFETCH, RING) * CHUNK, CHUNK)
            pltpu.async_copy(
                x_hbm.at[pl.ds(base + (c + PREFETCH) * CHUNK, CHUNK)],
                x_v.at[pl.ds(off2, CHUNK)], lsem)

        pltpu.async_copy(
            x_v.at[pl.ds(xoff, CHUNK)],
            out_hbm.at[pl.ds(base + c * CHUNK, CHUNK)], ssem)
        return carry

    lax.fori_loop(0, NCHUNK, chunk, 0)

    for _ in range(PREFETCH + 1):
        pltpu.make_async_copy(
            x_v.at[pl.ds(0, CHUNK)], out_hbm.at[pl.ds(base, CHUNK)],
            ssem).wait()


@jax.jit
def kernel(x, label, class_profile):
    mesh = plsc.VectorSubcoreMesh(core_axis_name="c", subcore_axis_name="s")
    run = pl.kernel(
        _gate_kernel,
        out_type=jax.ShapeDtypeStruct((BATCH, N_UNITS), jnp.float32),
        mesh=mesh,
        scratch_types=[
            pltpu.VMEM((B_PER_W,), jnp.int32),
            pltpu.VMEM((B_PER_W, N_UNITS), jnp.float32),
            pltpu.VMEM((RING * CHUNK, N_UNITS), jnp.float32),
            pltpu.VMEM_SHARED((N_CLASSES, N_UNITS), jnp.float32),
            pltpu.SemaphoreType.DMA,
            pltpu.SemaphoreType.DMA,
            pltpu.SemaphoreType.DMA,
        ],
    )
    return run(x, label, class_profile)


# speedup vs baseline: 1.0756x; 1.0756x over previous
import jax
import jax.numpy as jnp
from jax import lax
from jax.experimental import pallas as pl
from jax.experimental.pallas import tpu as pltpu
from jax.experimental.pallas import tpu_sc as plsc

N_CLASSES = 1000
N_UNITS = 128
BATCH = 16384

NC = 2
NS = 16
LANES = 16
NW = NC * NS
B_PER_W = BATCH // NW
CHUNK = 64
NCHUNK = B_PER_W // CHUNK
RING = 6
PREFETCH = 5


def _gate_kernel(x_hbm, label_hbm, table_hbm, out_hbm,
                 idx_v, rows_v, x_v, table_sh, gsem, lsem, ssem):
    sid = lax.axis_index("s")
    wid = sid * NC + lax.axis_index("c")
    base = wid * B_PER_W

    for c in range(PREFETCH):
        pltpu.async_copy(
            x_hbm.at[pl.ds(base + c * CHUNK, CHUNK)],
            x_v.at[pl.ds(c * CHUNK, CHUNK)], lsem)

    @pl.when(sid < 7)
    def _stage():
        r0 = sid * 128
        pltpu.sync_copy(table_hbm.at[pl.ds(r0, 128)],
                        table_sh.at[pl.ds(r0, 128)])

    @pl.when(sid == 7)
    def _stage_tail():
        pltpu.sync_copy(table_hbm.at[pl.ds(896, N_CLASSES - 896)],
                        table_sh.at[pl.ds(896, N_CLASSES - 896)])

    pltpu.sync_copy(label_hbm.at[pl.ds(base, B_PER_W)], idx_v)
    plsc.subcore_barrier()

    for c in range(NCHUNK):
        pltpu.async_copy(
            table_sh.at[idx_v.at[pl.ds(c * CHUNK, CHUNK)]],
            rows_v.at[pl.ds(c * CHUNK, CHUNK)], gsem)

    def chunk(c, carry):
        xoff = pl.multiple_of(lax.rem(c, RING) * CHUNK, CHUNK)
        roff = pl.multiple_of(c * CHUNK, CHUNK)
        pltpu.make_async_copy(
            table_sh.at[idx_v.at[pl.ds(roff, CHUNK)]],
            rows_v.at[pl.ds(roff, CHUNK)], gsem).wait()
        pltpu.make_async_copy(
            x_hbm.at[pl.ds(base, CHUNK)], x_v.at[pl.ds(xoff, CHUNK)], lsem).wait()

        @plsc.parallel_loop(0, CHUNK, unroll=4)
        def body(r):
            for j in range(N_UNITS // LANES):
                sl = pl.ds(j * LANES, LANES)
                x_v[xoff + r, sl] = x_v[xoff + r, sl] * rows_v[roff + r, sl]

        @pl.when(jnp.logical_and(c >= 1, c + PREFETCH < NCHUNK))
        def _drain_store():
            pltpu.make_async_copy(
                x_v.at[pl.ds(0, CHUNK)], out_hbm.at[pl.ds(base, CHUNK)],
                ssem).wait()

        @pl.when(c + PREFETCH < NCHUNK)
        def _next_load():
            off2 = pl.multiple_of(lax.rem(c + PREFETCH, RING) * CHUNK, CHUNK)
            pltpu.async_copy(
                x_hbm.at[pl.ds(base + (c + PREFETCH) * CHUNK, CHUNK)],
                x_v.at[pl.ds(off2, CHUNK)], lsem)

        pltpu.async_copy(
            x_v.at[pl.ds(xoff, CHUNK)],
            out_hbm.at[pl.ds(base + c * CHUNK, CHUNK)], ssem)
        return carry

    lax.fori_loop(0, NCHUNK, chunk, 0)

    for _ in range(PREFETCH + 1):
        pltpu.make_async_copy(
            x_v.at[pl.ds(0, CHUNK)], out_hbm.at[pl.ds(base, CHUNK)],
            ssem).wait()


@jax.jit
def kernel(x, label, class_profile):
    mesh = plsc.VectorSubcoreMesh(core_axis_name="c", subcore_axis_name="s")
    run = pl.kernel(
        _gate_kernel,
        out_type=jax.ShapeDtypeStruct((BATCH, N_UNITS), jnp.float32),
        mesh=mesh,
        scratch_types=[
            pltpu.VMEM((B_PER_W,), jnp.int32),
            pltpu.VMEM((B_PER_W, N_UNITS), jnp.float32),
            pltpu.VMEM((RING * CHUNK, N_UNITS), jnp.float32),
            pltpu.VMEM_SHARED((N_CLASSES, N_UNITS), jnp.float32),
            pltpu.SemaphoreType.DMA,
            pltpu.SemaphoreType.DMA,
            pltpu.SemaphoreType.DMA,
        ],
    )
    return run(x, label, class_profile)
